# Initial kernel scaffold; baseline (speedup 1.0000x reference)
import jax
import jax.numpy as jnp
from jax.experimental import pallas as pl


def kernel(x, edge_index, edge_attr, pos, W0_1, Wr_1, b1, W0_2, Wr_2, b2, lin1_W, lin1_b, lin2_W, lin2_b):
    raise NotImplementedError("write your pallas kernel here")


# speedup vs baseline: 7.9119x; 7.9119x over previous
import functools

import jax
import jax.numpy as jnp
from jax import lax
from jax.experimental import pallas as pl
from jax.experimental.pallas import tpu as pltpu
from jax.experimental.pallas import tpu_sc as plsc

N = 100000
K = 100
E = 3200000

NPAD = 96
NT = N + NPAD
RPT = NT // 16
W = 1024
GRP = W // 128
NWIN = 200
EPT = W * NWIN
EP = EPT * 16
NCH = 7
KP = NCH * 16


def _sc_body(*refs):
    tabs = refs[0:8]
    col3, row3, ea2, zz = refs[8:12]
    outs = refs[12:20]
    acc, cidx, sidx, eab, rows, gsem, ssem = refs[20:27]

    cid = lax.axis_index("c")
    sid = lax.axis_index("s")

    def run_task(table, out, gidx3, sidx3, scaled):
        base = sid * RPT
        pltpu.sync_copy(zz, acc.at[pl.ds(base, RPT)])
        plsc.subcore_barrier()

        def win_body(w, carry):
            wa = sid * NWIN + w
            pltpu.sync_copy(gidx3.at[wa], cidx)
            pltpu.sync_copy(sidx3.at[wa], sidx)
            if scaled:
                pltpu.sync_copy(ea2.at[wa], eab)
            gathers = [
                pltpu.async_copy(table.at[cidx.at[g]],
                                 rows.at[pl.ds(g * 128, 128)], gsem)
                for g in range(GRP)
            ]
            for c in gathers:
                c.wait()
            if scaled:
                dnums = lax.GatherDimensionNumbers(
                    offset_dims=(), collapsed_slice_dims=(0,),
                    start_index_map=(0,))

                def scale_body(g2, c2):
                    b0 = g2 * 16
                    e16 = eab[pl.ds(b0, 16)]
                    for j in range(16):
                        idx = jnp.full((16, 1), j, jnp.int32)
                        ev = lax.gather(
                            e16, idx, dnums, slice_sizes=(1,),
                            mode=lax.GatherScatterMode.PROMISE_IN_BOUNDS)
                        rows[b0 + j] = rows[b0 + j] * ev
                    return c2
                lax.fori_loop(0, W // 16, scale_body, 0, unroll=False)
            scatters = [
                pltpu.async_copy(rows.at[pl.ds(g * 128, 128)],
                                 acc.at[sidx.at[g]], ssem, add=True)
                for g in range(GRP)
            ]
            for c in scatters:
                c.wait()
            return carry

        lax.fori_loop(0, NWIN, win_body, 0, unroll=False)
        plsc.subcore_barrier()
        pltpu.sync_copy(acc.at[pl.ds(base, RPT)], out.at[pl.ds(base, RPT)])

    for t in range(8):
        owner = 0 if t < 4 else 1
        if t < 7:
            args = (tabs[t], outs[t], col3, row3, True)
        else:
            args = (tabs[7], outs[7], row3, col3, False)

        @pl.when(cid == owner)
        def _(args=args):
            run_task(*args)


@functools.partial(jax.jit, static_argnums=())
def _sc_pass(t0, t1, t2, t3, t4, t5, t6, xp, col3, row3, ea2, zz):
    mesh = plsc.VectorSubcoreMesh(core_axis_name="c", subcore_axis_name="s")
    fn = pl.kernel(
        _sc_body,
        out_type=[jax.ShapeDtypeStruct((NT, 16), jnp.float32)] * 8,
        mesh=mesh,
        scratch_types=[
            pltpu.VMEM_SHARED((NT, 16), jnp.float32),
            pltpu.VMEM((GRP, 128), jnp.int32),
            pltpu.VMEM((GRP, 128), jnp.int32),
            pltpu.VMEM((W,), jnp.float32),
            pltpu.VMEM((W, 16), jnp.float32),
            pltpu.SemaphoreType.DMA,
            pltpu.SemaphoreType.DMA,
        ],
        compiler_params=pltpu.CompilerParams(use_tc_tiling_on_sc=False),
    )
    return fn(t0, t1, t2, t3, t4, t5, t6, xp, col3, row3, ea2, zz)


BLK = 2000
GRID = N // BLK


def _tc_body(xp, sxc, posz, s0, s1, s2, s3, s4, s5, s6,
             w0e, wre, b1, w02, wr2, b2, l1w, l1b, l2w, l2b,
             out_ref, hx_acc, adj_acc):
    i = pl.program_id(0)

    @pl.when(i == 0)
    def _():
        hx_acc[...] = jnp.zeros_like(hx_acc)
        adj_acc[...] = jnp.zeros_like(adj_acc)

    xpb = xp[...]
    sxb = sxc[...]
    posb = posz[...]
    sp = jnp.concatenate([s0[...], s1[...], s2[...], s3[...],
                          s4[...], s5[...], s6[...]], axis=1)

    cnt = jnp.maximum(sxb[:, 5:6], 1.0)
    aggw = jnp.dot(sxb, w0e[...], preferred_element_type=jnp.float32) / cnt
    h = jnp.maximum(
        aggw + jnp.dot(xpb, wre[...], preferred_element_type=jnp.float32)
        + b1[...], 0.0)

    dn = (((0,), (0,)), ((), ()))
    hx_acc[...] += lax.dot_general(posb, h, dn,
                                   preferred_element_type=jnp.float32)
    adj_acc[...] += lax.dot_general(posb, sp, dn,
                                    preferred_element_type=jnp.float32)

    @pl.when(i == GRID - 1)
    def _():
        adj = adj_acc[...]
        hx = hx_acc[...]
        m = (adj != 0.0).astype(jnp.float32)
        hxw = jnp.dot(hx, w02[...], preferred_element_type=jnp.float32)
        s2m = lax.dot_general(m, hxw, dn,
                              preferred_element_type=jnp.float32)
        cnt2 = jnp.sum(m, axis=0)[:, None]
        agg2 = s2m / jnp.maximum(cnt2, 1.0)
        h2 = jnp.maximum(
            agg2 + jnp.dot(hx, wr2[...], preferred_element_type=jnp.float32)
            + b2[...], 0.0)
        valid = (lax.broadcasted_iota(jnp.int32, (KP, 1), 0)
                 < K).astype(jnp.float32)
        pooled = jnp.sum(h2 * valid, axis=0, keepdims=True) / float(K)
        z = jnp.dot(pooled, l1w[...], preferred_element_type=jnp.float32) \
            + l1b[...]
        x4 = jnp.where(z >= 0.0, z, 0.1 * z)
        out_ref[...] = jnp.dot(x4, l2w[...],
                               preferred_element_type=jnp.float32) + l2b[...]


def _tc_pass(xp, sxc, posz, sps, w0e, wre, b1, w02, wr2, b2,
             l1w, l1b, l2w, l2b, interpret=False):
    node_spec16 = pl.BlockSpec((BLK, 16), lambda i: (i, 0))
    node_spec112 = pl.BlockSpec((BLK, KP), lambda i: (i, 0))
    wts = (w0e, wre, b1, w02, wr2, b2, l1w, l1b, l2w, l2b)
    in_specs = ([node_spec16, node_spec16, node_spec112]
                + [node_spec16] * 7
                + [pl.BlockSpec(w.shape, lambda i: (0, 0)) for w in wts])
    return pl.pallas_call(
        _tc_body,
        grid=(GRID,),
        in_specs=in_specs,
        out_specs=pl.BlockSpec((1, 2), lambda i: (0, 0)),
        out_shape=jax.ShapeDtypeStruct((1, 2), jnp.float32),
        scratch_shapes=[pltpu.VMEM((KP, 16), jnp.float32),
                        pltpu.VMEM((KP, KP), jnp.float32)],
        interpret=interpret,
    )(xp, sxc, posz, *sps, w0e, wre, b1, w02, wr2, b2, l1w, l1b, l2w, l2b)


def kernel(x, edge_index, edge_attr, pos, W0_1, Wr_1, b1, W0_2, Wr_2, b2,
           lin1_W, lin1_b, lin2_W, lin2_b):
    row = edge_index[0]
    col = edge_index[1]
    ea = edge_attr[:, 0]

    pad = EP - E
    pidx = (jnp.arange(pad, dtype=jnp.int32) % NPAD) + N
    row_p = jnp.concatenate([row, pidx]).reshape(EP // W, GRP, 128)
    col_p = jnp.concatenate([col, pidx]).reshape(EP // W, GRP, 128)
    ea_p = jnp.concatenate([ea, jnp.zeros((pad,), jnp.float32)])
    ea2 = ea_p.reshape(EP // W, W)

    posz = jnp.pad(pos, ((0, NPAD), (0, KP - K)))
    tabs = [posz[:, q * 16:(q + 1) * 16] for q in range(NCH)]
    xp = jnp.zeros((NT, 16), jnp.float32)
    xp = xp.at[:N, :5].set(x).at[:N, 5].set(1.0)
    zz = jnp.zeros((RPT, 16), jnp.float32)

    outs = _sc_pass(*tabs, xp, col_p, row_p, ea2, zz)
    sps, sxc = list(outs[:7]), outs[7]

    w0e = jnp.zeros((16, 16), jnp.float32).at[:5].set(W0_1)
    wre = jnp.zeros((16, 16), jnp.float32).at[:5].set(Wr_1)
    out2 = _tc_pass(xp, sxc, posz, sps, w0e, wre, b1.reshape(1, 16),
                    W0_2, Wr_2, b2.reshape(1, 32), lin1_W,
                    lin1_b.reshape(1, 8), lin2_W, lin2_b.reshape(1, 2))
    return out2.reshape(2)


# speedup vs baseline: 20.1781x; 2.5503x over previous
import jax
import jax.numpy as jnp
from jax import lax
from jax.experimental import pallas as pl
from jax.experimental.pallas import tpu as pltpu
from jax.experimental.pallas import tpu_sc as plsc

N = 100000
K = 100
E = 3200000

NPAD = 96
NT = N + NPAD
RPT = NT // 16
W = 640
GRP = W // 128
EP = 3276800
NWTOT = EP // W
NWIN_SP = EP // W // 16
NWIN_SX = NWIN_SP // 2
NCH = 4
KP = NCH * 32


def _sc_body(*refs):
    (pc0, pc1, pc2, pc3, xpb, col3, row3, rowm3, zz) = refs[0:9]
    outs = refs[9:15]
    (acc, cidx0, sidx0, rows0, cidx1, sidx1, rows1,
     isem0, gsem0, ssem0, isem1, gsem1, ssem1) = refs[15:28]

    cid = lax.axis_index("c")
    sid = lax.axis_index("s")

    bufs = [(cidx0, sidx0, rows0, isem0, gsem0, ssem0),
            (cidx1, sidx1, rows1, isem1, gsem1, ssem1)]

    def run_task(table, out, gidx3, sidx3, nwin, wbase):
        pltpu.sync_copy(zz, acc.at[pl.ds(sid * RPT, RPT)])
        plsc.subcore_barrier()

        def fetch_idx(w, b):
            ci, si, rw, isem, gs, ss = bufs[b]
            pltpu.async_copy(gidx3.at[w], ci, isem)
            pltpu.async_copy(sidx3.at[w], si, isem)

        def drain_idx(b):
            ci, si, rw, isem, gs, ss = bufs[b]
            pltpu.make_async_copy(col3.at[0], ci, isem).wait()
            pltpu.make_async_copy(col3.at[0], si, isem).wait()

        def fire_gathers(b):
            ci, si, rw, isem, gs, ss = bufs[b]
            for g in range(GRP):
                pltpu.async_copy(table.at[ci.at[g]],
                                 rw.at[pl.ds(g * 128, 128)], gs)

        def drain_gathers(b):
            ci, si, rw, isem, gs, ss = bufs[b]
            for g in range(GRP):
                pltpu.make_async_copy(table.at[ci.at[g]],
                                      rw.at[pl.ds(g * 128, 128)], gs).wait()

        def fire_scatters(b):
            ci, si, rw, isem, gs, ss = bufs[b]
            for g in range(GRP):
                pltpu.async_copy(rw.at[pl.ds(g * 128, 128)],
                                 acc.at[si.at[g]], ss, add=True)

        def drain_scatters(b):
            ci, si, rw, isem, gs, ss = bufs[b]
            for g in range(GRP):
                pltpu.make_async_copy(rw.at[pl.ds(g * 128, 128)],
                                      acc.at[si.at[g]], ss).wait()

        half = nwin // 2
        fetch_idx(wbase, 0)
        drain_idx(0)
        fire_gathers(0)
        fetch_idx(wbase + 1, 1)

        def ib(i, c):
            w = wbase + 2 * i
            drain_idx(1)

            @pl.when(i > 0)
            def _():
                drain_scatters(1)
            fire_gathers(1)

            drain_gathers(0)

            @pl.when(i < half - 1)
            def _():
                fetch_idx(w + 2, 0)
            fire_scatters(0)

            @pl.when(i < half - 1)
            def _():
                drain_scatters(0)
                drain_idx(0)
                fire_gathers(0)

            drain_gathers(1)

            @pl.when(i < half - 1)
            def _():
                fetch_idx(w + 3, 1)
            fire_scatters(1)
            return c

        lax.fori_loop(0, half, ib, 0, unroll=False)
        drain_scatters(0)
        drain_scatters(1)
        plsc.subcore_barrier()
        pltpu.sync_copy(acc.at[pl.ds(sid * RPT, RPT)],
                        out.at[pl.ds(sid * RPT, RPT)])

    tasks = [
        (0, pc0, outs[0], col3, rowm3, NWIN_SP, sid * NWIN_SP),
        (0, pc1, outs[1], col3, rowm3, NWIN_SP, sid * NWIN_SP),
        (1, pc2, outs[2], col3, rowm3, NWIN_SP, sid * NWIN_SP),
        (1, pc3, outs[3], col3, rowm3, NWIN_SP, sid * NWIN_SP),
        (0, xpb, outs[4], row3, col3, NWIN_SX, sid * NWIN_SX),
        (1, xpb, outs[5], row3, col3, NWIN_SX, NWTOT // 2 + sid * NWIN_SX),
    ]
    for owner, table, out, g3, s3, nwin, wbase in tasks:
        @pl.when(cid == owner)
        def _(table=table, out=out, g3=g3, s3=s3, nwin=nwin, wbase=wbase):
            run_task(table, out, g3, s3, nwin, wbase)


def _sc_pass(pc0, pc1, pc2, pc3, xpb, col3, row3, rowm3, zz):
    mesh = plsc.VectorSubcoreMesh(core_axis_name="c", subcore_axis_name="s")
    fn = pl.kernel(
        _sc_body,
        out_type=[jax.ShapeDtypeStruct((NT, 32), jnp.bfloat16)] * 6,
        mesh=mesh,
        scratch_types=[
            pltpu.VMEM_SHARED((NT, 32), jnp.bfloat16),
            pltpu.VMEM((GRP, 128), jnp.int32),
            pltpu.VMEM((GRP, 128), jnp.int32),
            pltpu.VMEM((W, 32), jnp.bfloat16),
            pltpu.VMEM((GRP, 128), jnp.int32),
            pltpu.VMEM((GRP, 128), jnp.int32),
            pltpu.VMEM((W, 32), jnp.bfloat16),
            pltpu.SemaphoreType.DMA,
            pltpu.SemaphoreType.DMA,
            pltpu.SemaphoreType.DMA,
            pltpu.SemaphoreType.DMA,
            pltpu.SemaphoreType.DMA,
            pltpu.SemaphoreType.DMA,
        ],
        compiler_params=pltpu.CompilerParams(use_tc_tiling_on_sc=False),
    )
    return fn(pc0, pc1, pc2, pc3, xpb, col3, row3, rowm3, zz)


BLK = 2000
GRID = N // BLK


def _tc_body(xpf, sxa, sxb, posz, s0, s1, s2, s3,
             w0e, wre, b1, w02, wr2, b2, l1w, l1b, l2w, l2b,
             out_ref, hx_acc, adj_acc):
    i = pl.program_id(0)

    @pl.when(i == 0)
    def _():
        hx_acc[...] = jnp.zeros_like(hx_acc)
        adj_acc[...] = jnp.zeros_like(adj_acc)

    xb = xpf[...]
    sx = (sxa[...].astype(jnp.float32)
          + sxb[...].astype(jnp.float32))
    posb = posz[...]
    sp = jnp.concatenate([s0[...], s1[...], s2[...], s3[...]],
                         axis=1).astype(jnp.float32)

    cnt = jnp.maximum(sx[:, 5:6], 1.0)
    aggw = jnp.dot(sx, w0e[...], preferred_element_type=jnp.float32) / cnt
    h = jnp.maximum(
        aggw + jnp.dot(xb, wre[...], preferred_element_type=jnp.float32)
        + b1[...], 0.0)

    dn = (((0,), (0,)), ((), ()))
    hx_acc[...] += lax.dot_general(posb, h, dn,
                                   preferred_element_type=jnp.float32)
    adj_acc[...] += lax.dot_general(posb, sp, dn,
                                    preferred_element_type=jnp.float32)

    @pl.when(i == GRID - 1)
    def _():
        adj = adj_acc[...]
        hx = hx_acc[...]
        m = (adj != 0.0).astype(jnp.float32)
        hxw = jnp.dot(hx, w02[...], preferred_element_type=jnp.float32)
        s2m = lax.dot_general(m, hxw, dn,
                              preferred_element_type=jnp.float32)
        cnt2 = jnp.sum(m, axis=0)[:, None]
        agg2 = s2m / jnp.maximum(cnt2, 1.0)
        h2 = jnp.maximum(
            agg2 + jnp.dot(hx, wr2[...], preferred_element_type=jnp.float32)
            + b2[...], 0.0)
        valid = (lax.broadcasted_iota(jnp.int32, (KP, 1), 0)
                 < K).astype(jnp.float32)
        pooled = jnp.sum(h2 * valid, axis=0, keepdims=True) / float(K)
        z = jnp.dot(pooled, l1w[...], preferred_element_type=jnp.float32) \
            + l1b[...]
        x4 = jnp.where(z >= 0.0, z, 0.1 * z)
        out_ref[...] = jnp.dot(x4, l2w[...],
                               preferred_element_type=jnp.float32) + l2b[...]


def _tc_pass(xpf, sxa, sxb, posz, sps, w0e, wre, b1, w02, wr2, b2,
             l1w, l1b, l2w, l2b, interpret=False):
    ns16 = pl.BlockSpec((BLK, 16), lambda i: (i, 0))
    ns32 = pl.BlockSpec((BLK, 32), lambda i: (i, 0))
    ns128 = pl.BlockSpec((BLK, KP), lambda i: (i, 0))
    wts = (w0e, wre, b1, w02, wr2, b2, l1w, l1b, l2w, l2b)
    in_specs = ([ns16, ns32, ns32, ns128] + [ns32] * 4
                + [pl.BlockSpec(w.shape, lambda i: (0, 0)) for w in wts])
    return pl.pallas_call(
        _tc_body,
        grid=(GRID,),
        in_specs=in_specs,
        out_specs=pl.BlockSpec((1, 2), lambda i: (0, 0)),
        out_shape=jax.ShapeDtypeStruct((1, 2), jnp.float32),
        scratch_shapes=[pltpu.VMEM((KP, 16), jnp.float32),
                        pltpu.VMEM((KP, KP), jnp.float32)],
        interpret=interpret,
    )(xpf, sxa, sxb, posz, *sps, *wts)


def kernel(x, edge_index, edge_attr, pos, W0_1, Wr_1, b1, W0_2, Wr_2, b2,
           lin1_W, lin1_b, lin2_W, lin2_b):
    row = edge_index[0]
    col = edge_index[1]
    ea = edge_attr[:, 0]

    pad = EP - E
    pidx = (jnp.arange(pad, dtype=jnp.int32) % NPAD) + N
    dummy = (jnp.arange(E, dtype=jnp.int32) % NPAD) + N
    rowm = jnp.where(ea != 0.0, row, dummy)
    row_p = jnp.concatenate([row, pidx]).reshape(NWTOT, GRP, 128)
    col_p = jnp.concatenate([col, pidx]).reshape(NWTOT, GRP, 128)
    rowm_p = jnp.concatenate([rowm, pidx]).reshape(NWTOT, GRP, 128)

    posz = jnp.pad(pos, ((0, NPAD), (0, KP - K)))
    posbf = posz.astype(jnp.bfloat16)
    tabs = [posbf[:, q * 32:(q + 1) * 32] for q in range(NCH)]
    xpb = jnp.zeros((NT, 32), jnp.bfloat16)
    xpb = xpb.at[:N, :5].set(x.astype(jnp.bfloat16)).at[:N, 5].set(1.0)
    xpf = jnp.zeros((NT, 16), jnp.float32)
    xpf = xpf.at[:N, :5].set(x)
    zz = jnp.zeros((RPT, 32), jnp.bfloat16)

    outs = _sc_pass(*tabs, xpb, col_p, row_p, rowm_p, zz)
    sps, sxa, sxb = list(outs[:4]), outs[4], outs[5]

    w0e = jnp.zeros((32, 16), jnp.float32).at[:5].set(W0_1)
    wre = jnp.zeros((16, 16), jnp.float32).at[:5].set(Wr_1)
    out2 = _tc_pass(xpf, sxa, sxb, posz, sps, w0e, wre, b1.reshape(1, 16),
                    W0_2, Wr_2, b2.reshape(1, 32), lin1_W,
                    lin1_b.reshape(1, 8), lin2_W, lin2_b.reshape(1, 2))
    return out2.reshape(2)


# speedup vs baseline: 22.0105x; 1.0908x over previous
import jax
import jax.numpy as jnp
from jax import lax
from jax.experimental import pallas as pl
from jax.experimental.pallas import tpu as pltpu
from jax.experimental.pallas import tpu_sc as plsc

N = 100000
K = 100
E = 3200000

NPAD = 96
NT = N + NPAD
RPT = NT // 16
W = 640
GRP = W // 128
EP = 3276800
NWTOT = EP // W
NWIN_SP = EP // W // 16
NWIN_SX = NWIN_SP // 2
NCH = 4
KP = NCH * 32


def _sc_body(*refs):
    (pc0, pc1, pc2, pc3, xpb, col2, row2, rowm2, zz) = refs[0:9]
    outs = refs[9:15]
    (acc, cidx0, sidx0, rows0, cidx1, sidx1, rows1,
     isem0, gsem0, ssem0, isem1, gsem1, ssem1) = refs[15:28]

    cid = lax.axis_index("c")
    sid = lax.axis_index("s")

    bufs = [(cidx0, sidx0, rows0, isem0, gsem0, ssem0),
            (cidx1, sidx1, rows1, isem1, gsem1, ssem1)]

    def run_task(table, out, gidx2, sidx2, nwin, wbase):
        pltpu.sync_copy(zz, acc.at[pl.ds(sid * RPT, RPT)])
        plsc.subcore_barrier()

        def fetch_idx(w, b):
            ci, si, rw, isem, gs, ss = bufs[b]
            pltpu.async_copy(gidx2.at[pl.ds(w * GRP, GRP)], ci, isem)
            pltpu.async_copy(sidx2.at[pl.ds(w * GRP, GRP)], si, isem)

        def drain_idx(b):
            ci, si, rw, isem, gs, ss = bufs[b]
            pltpu.make_async_copy(col2.at[pl.ds(0, GRP)], ci, isem).wait()
            pltpu.make_async_copy(col2.at[pl.ds(0, GRP)], si, isem).wait()

        def fire_gathers(b):
            ci, si, rw, isem, gs, ss = bufs[b]
            for g in range(GRP):
                pltpu.async_copy(table.at[ci.at[g]],
                                 rw.at[pl.ds(g * 128, 128)], gs)

        def drain_gathers(b):
            ci, si, rw, isem, gs, ss = bufs[b]
            for g in range(GRP):
                pltpu.make_async_copy(table.at[ci.at[g]],
                                      rw.at[pl.ds(g * 128, 128)], gs).wait()

        def fire_scatters(b):
            ci, si, rw, isem, gs, ss = bufs[b]
            for g in range(GRP):
                pltpu.async_copy(rw.at[pl.ds(g * 128, 128)],
                                 acc.at[si.at[g]], ss, add=True)

        def drain_scatters(b):
            ci, si, rw, isem, gs, ss = bufs[b]
            for g in range(GRP):
                pltpu.make_async_copy(rw.at[pl.ds(g * 128, 128)],
                                      acc.at[si.at[g]], ss).wait()

        half = nwin // 2
        fetch_idx(wbase, 0)
        drain_idx(0)
        fire_gathers(0)
        fetch_idx(wbase + 1, 1)

        def ib(i, c):
            w = wbase + 2 * i
            drain_idx(1)

            @pl.when(i > 0)
            def _():
                drain_scatters(1)
            fire_gathers(1)

            drain_gathers(0)

            @pl.when(i < half - 1)
            def _():
                fetch_idx(w + 2, 0)
            fire_scatters(0)

            @pl.when(i < half - 1)
            def _():
                drain_scatters(0)
                drain_idx(0)
                fire_gathers(0)

            drain_gathers(1)

            @pl.when(i < half - 1)
            def _():
                fetch_idx(w + 3, 1)
            fire_scatters(1)
            return c

        lax.fori_loop(0, half, ib, 0, unroll=False)
        drain_scatters(0)
        drain_scatters(1)
        plsc.subcore_barrier()
        pltpu.sync_copy(acc.at[pl.ds(sid * RPT, RPT)],
                        out.at[pl.ds(sid * RPT, RPT)])

    tasks = [
        (0, pc0, outs[0], col2, rowm2, NWIN_SP, sid * NWIN_SP),
        (0, pc1, outs[1], col2, rowm2, NWIN_SP, sid * NWIN_SP),
        (1, pc2, outs[2], col2, rowm2, NWIN_SP, sid * NWIN_SP),
        (1, pc3, outs[3], col2, rowm2, NWIN_SP, sid * NWIN_SP),
        (0, xpb, outs[4], row2, col2, NWIN_SX, sid * NWIN_SX),
        (1, xpb, outs[5], row2, col2, NWIN_SX, NWTOT // 2 + sid * NWIN_SX),
    ]
    for owner, table, out, g3, s3, nwin, wbase in tasks:
        @pl.when(cid == owner)
        def _(table=table, out=out, g3=g3, s3=s3, nwin=nwin, wbase=wbase):
            run_task(table, out, g3, s3, nwin, wbase)


def _sc_pass(pc0, pc1, pc2, pc3, xpb, col3, row3, rowm3, zz):
    mesh = plsc.VectorSubcoreMesh(core_axis_name="c", subcore_axis_name="s")
    fn = pl.kernel(
        _sc_body,
        out_type=[jax.ShapeDtypeStruct((NT, 32), jnp.bfloat16)] * 6,
        mesh=mesh,
        scratch_types=[
            pltpu.VMEM_SHARED((NT, 32), jnp.bfloat16),
            pltpu.VMEM((GRP, 128), jnp.int32),
            pltpu.VMEM((GRP, 128), jnp.int32),
            pltpu.VMEM((W, 32), jnp.bfloat16),
            pltpu.VMEM((GRP, 128), jnp.int32),
            pltpu.VMEM((GRP, 128), jnp.int32),
            pltpu.VMEM((W, 32), jnp.bfloat16),
            pltpu.SemaphoreType.DMA,
            pltpu.SemaphoreType.DMA,
            pltpu.SemaphoreType.DMA,
            pltpu.SemaphoreType.DMA,
            pltpu.SemaphoreType.DMA,
            pltpu.SemaphoreType.DMA,
        ],
        compiler_params=pltpu.CompilerParams(use_tc_tiling_on_sc=False),
    )
    return fn(pc0, pc1, pc2, pc3, xpb, col3, row3, rowm3, zz)


EB = 200
ER = E // 128
PR = EP // 128


def _prep_body(rowr, colr, ear, colo, rowo, rowmo):
    i = pl.program_id(0)
    r = lax.broadcasted_iota(jnp.int32, (EB, 128), 0) + i * EB
    lane = lax.broadcasted_iota(jnp.int32, (EB, 128), 1)
    dummy = N + lax.rem(r * 128 + lane, NPAD)
    real = r < ER
    colo[...] = jnp.where(real, colr[...], dummy)
    rowo[...] = jnp.where(real, rowr[...], dummy)
    rowmo[...] = jnp.where(real & (ear[...] != 0.0), rowr[...], dummy)


def _prep_pass(row2, col2, ea2):
    inspec = pl.BlockSpec((EB, 128), lambda i: (jnp.minimum(i, ER // EB - 1), 0))
    outspec = pl.BlockSpec((EB, 128), lambda i: (i, 0))
    return pl.pallas_call(
        _prep_body,
        grid=(PR // EB,),
        in_specs=[inspec, inspec, inspec],
        out_specs=[outspec, outspec, outspec],
        out_shape=[jax.ShapeDtypeStruct((PR, 128), jnp.int32)] * 3,
    )(row2, col2, ea2)


BLK = 2000
GRID = N // BLK


def _tc_body(xr, sxa, sxb, posz, s0, s1, s2, s3,
             w0e, wre, b1, w02, wr2, b2, l1w, l1b, l2w, l2b,
             out_ref, hx_acc, adj_acc):
    i = pl.program_id(0)

    @pl.when(i == 0)
    def _():
        hx_acc[...] = jnp.zeros_like(hx_acc)
        adj_acc[...] = jnp.zeros_like(adj_acc)

    xb = xr[...]
    sx = (sxa[...].astype(jnp.float32)
          + sxb[...].astype(jnp.float32))
    posb = posz[...]
    sp = jnp.concatenate([s0[...], s1[...], s2[...], s3[...]],
                         axis=1).astype(jnp.float32)

    cnt = jnp.maximum(sx[:, 5:6], 1.0)
    aggw = jnp.dot(sx, w0e[...], preferred_element_type=jnp.float32) / cnt
    h = jnp.maximum(
        aggw + jnp.dot(xb, wre[...], preferred_element_type=jnp.float32)
        + b1[...], 0.0)

    dn = (((0,), (0,)), ((), ()))
    hx_acc[...] += lax.dot_general(posb, h, dn,
                                   preferred_element_type=jnp.float32)
    adj_acc[...] += lax.dot_general(posb, sp, dn,
                                    preferred_element_type=jnp.float32)

    @pl.when(i == GRID - 1)
    def _():
        adj = adj_acc[...]
        hx = hx_acc[...]
        m = (adj != 0.0).astype(jnp.float32)
        hxw = jnp.dot(hx, w02[...], preferred_element_type=jnp.float32)
        s2m = lax.dot_general(m, hxw, dn,
                              preferred_element_type=jnp.float32)
        cnt2 = jnp.sum(m, axis=0)[:, None]
        agg2 = s2m / jnp.maximum(cnt2, 1.0)
        h2 = jnp.maximum(
            agg2 + jnp.dot(hx, wr2[...], preferred_element_type=jnp.float32)
            + b2[...], 0.0)
        valid = (lax.broadcasted_iota(jnp.int32, (KP, 1), 0)
                 < K).astype(jnp.float32)
        pooled = jnp.sum(h2 * valid, axis=0, keepdims=True) / float(K)
        z = jnp.dot(pooled, l1w[...], preferred_element_type=jnp.float32) \
            + l1b[...]
        x4 = jnp.where(z >= 0.0, z, 0.1 * z)
        out_ref[...] = jnp.dot(x4, l2w[...],
                               preferred_element_type=jnp.float32) + l2b[...]


def _tc_pass(xr, sxa, sxb, posz, sps, w0e, wre, b1, w02, wr2, b2,
             l1w, l1b, l2w, l2b, interpret=False):
    ns5 = pl.BlockSpec((BLK, 5), lambda i: (i, 0))
    ns32 = pl.BlockSpec((BLK, 32), lambda i: (i, 0))
    ns128 = pl.BlockSpec((BLK, KP), lambda i: (i, 0))
    wts = (w0e, wre, b1, w02, wr2, b2, l1w, l1b, l2w, l2b)
    in_specs = ([ns5, ns32, ns32, ns128] + [ns32] * 4
                + [pl.BlockSpec(w.shape, lambda i: (0, 0)) for w in wts])
    return pl.pallas_call(
        _tc_body,
        grid=(GRID,),
        in_specs=in_specs,
        out_specs=pl.BlockSpec((1, 2), lambda i: (0, 0)),
        out_shape=jax.ShapeDtypeStruct((1, 2), jnp.float32),
        scratch_shapes=[pltpu.VMEM((KP, 16), jnp.float32),
                        pltpu.VMEM((KP, KP), jnp.float32)],
        interpret=interpret,
    )(xr, sxa, sxb, posz, *sps, *wts)


def kernel(x, edge_index, edge_attr, pos, W0_1, Wr_1, b1, W0_2, Wr_2, b2,
           lin1_W, lin1_b, lin2_W, lin2_b):
    row2 = edge_index[0].reshape(ER, 128)
    col2 = edge_index[1].reshape(ER, 128)
    ea2 = edge_attr.reshape(ER, 128)

    col_p, row_p, rowm_p = _prep_pass(row2, col2, ea2)

    posz = jnp.pad(pos, ((0, NPAD), (0, KP - K)))
    posbf = posz.astype(jnp.bfloat16)
    tabs = [posbf[:, q * 32:(q + 1) * 32] for q in range(NCH)]
    xpb = jnp.pad(
        jnp.concatenate([x, jnp.ones((N, 1), jnp.float32)], axis=1),
        ((0, NPAD), (0, 26))).astype(jnp.bfloat16)
    zz = jnp.zeros((RPT, 32), jnp.bfloat16)

    outs = _sc_pass(*tabs, xpb, col_p, row_p, rowm_p, zz)
    sps, sxa, sxb = list(outs[:4]), outs[4], outs[5]

    w0e = jnp.zeros((32, 16), jnp.float32).at[:5].set(W0_1)
    out2 = _tc_pass(x, sxa, sxb, posz, sps, w0e, Wr_1, b1.reshape(1, 16),
                    W0_2, Wr_2, b2.reshape(1, 32), lin1_W,
                    lin1_b.reshape(1, 8), lin2_W, lin2_b.reshape(1, 2))
    return out2.reshape(2)
